# Initial kernel scaffold; baseline (speedup 1.0000x reference)
import jax
import jax.numpy as jnp
from jax.experimental import pallas as pl


def kernel(edge_index, edge_weight, emb_users, emb_items, W, b):
    raise NotImplementedError("write your pallas kernel here")


# speedup vs baseline: 7.9996x; 7.9996x over previous
import functools
import jax
import jax.numpy as jnp
from jax import lax
from jax.experimental import pallas as pl
from jax.experimental.pallas import tpu as pltpu
from jax.experimental.pallas import tpu_sc as plsc

NU = 25000
NI = 25000
NN = NU + NI
EE = 800000
DD = 64
NW = 32
CK = 128
CH = 196
EW_ = CK * CH
EP = NW * EW_
NP_ = 50176
NT = 3136
HALF = 25000
HP = 25088
TR = 1568

_mesh = plsc.VectorSubcoreMesh(core_axis_name="c", subcore_axis_name="s")
_f32 = jnp.float32
_i32 = jnp.int32


def _deg_body(col_hbm, ew_hbm, out_hbm, colv, ewv, zbuf, deg_sp):
    c = lax.axis_index("c")
    s = lax.axis_index("s")
    w = s * 2 + c

    def zb(i, _):
        zbuf[pl.ds(i * 16, 16)] = jnp.zeros((16,), _f32)
        return 0
    lax.fori_loop(0, NT // 16, zb, 0, unroll=8)
    pltpu.sync_copy(zbuf, deg_sp.at[pl.ds(s * NT, NT)])
    plsc.subcore_barrier()

    pltpu.sync_copy(col_hbm.at[w], colv)
    pltpu.sync_copy(ew_hbm.at[w], ewv)

    def chunk(j, _):
        pltpu.sync_copy(ewv.at[j], deg_sp.at[colv.at[j]], add=True)
        return 0
    lax.fori_loop(0, CH, chunk, 0)
    plsc.subcore_barrier()
    pltpu.sync_copy(deg_sp.at[pl.ds(s * NT, NT)], zbuf)
    pltpu.sync_copy(zbuf, out_hbm.at[pl.ds(c * NP_ + s * NT, NT)])


_deg_kernel = functools.partial(
    pl.kernel,
    out_type=jax.ShapeDtypeStruct((2 * NP_,), _f32),
    mesh=_mesh,
    compiler_params=pltpu.CompilerParams(needs_layout_passes=False),
    scratch_types=[
        pltpu.VMEM((CH, CK), _i32),
        pltpu.VMEM((CH, CK), _f32),
        pltpu.VMEM((NT,), _f32),
        pltpu.VMEM_SHARED((NP_,), _f32),
    ],
)(_deg_body)


def _dis_body(d0, d1, o):
    d = d0[...] + d1[...]
    o[...] = jnp.where(d > 0.0, lax.rsqrt(d), 0.0)


QN = EW_ // 2


def _norm_body(dis_hbm, row_hbm, col_hbm, ew_hbm, norm_hbm, lc_hbm,
               disv, rowq, colq, ewq, normq, lc0q, lc1q):
    c = lax.axis_index("c")
    s = lax.axis_index("s")
    w = s * 2 + c
    pltpu.sync_copy(dis_hbm, disv)
    for h in range(2):
        off = w * EW_ + h * QN
        pltpu.sync_copy(row_hbm.at[pl.ds(off, QN)], rowq)
        pltpu.sync_copy(col_hbm.at[pl.ds(off, QN)], colq)
        pltpu.sync_copy(ew_hbm.at[pl.ds(off, QN)], ewq)

        def step(t, _):
            i = t * 16
            rv = rowq[pl.ds(i, 16)]
            cv = colq[pl.ds(i, 16)]
            ev = ewq[pl.ds(i, 16)]
            a = plsc.load_gather(disv, [rv])
            b = plsc.load_gather(disv, [cv])
            normq[pl.ds(i, 16)] = a * b * ev
            dummy = jnp.full((16,), HALF, _i32)
            lc0q[pl.ds(i, 16)] = jnp.where(cv < HALF, cv, dummy)
            l1 = cv - HALF
            ok1 = (l1 >= 0) & (l1 < HALF)
            lc1q[pl.ds(i, 16)] = jnp.where(ok1, l1, dummy)
            return 0
        lax.fori_loop(0, QN // 16, step, 0, unroll=2)
        pltpu.sync_copy(normq, norm_hbm.at[pl.ds(off, QN)])
        pltpu.sync_copy(lc0q, lc_hbm.at[pl.ds(off, QN)])
        pltpu.sync_copy(lc1q, lc_hbm.at[pl.ds(EP + off, QN)])


_norm_kernel = functools.partial(
    pl.kernel,
    out_type=(jax.ShapeDtypeStruct((EP,), _f32),
              jax.ShapeDtypeStruct((2 * EP,), _i32)),
    mesh=_mesh,
    compiler_params=pltpu.CompilerParams(needs_layout_passes=False),
    scratch_types=[
        pltpu.VMEM((NP_,), _f32),
        pltpu.VMEM((QN,), _i32),
        pltpu.VMEM((QN,), _i32),
        pltpu.VMEM((QN,), _f32),
        pltpu.VMEM((QN,), _f32),
        pltpu.VMEM((QN,), _i32),
        pltpu.VMEM((QN,), _i32),
    ],
)(_norm_body)


GC = 7


def _layer_body(x_hbm, row_hbm, lc_hbm, norm_hbm, out_hbm,
                rowv, lcv, normv, rows_v, sem, out_sp):
    c = lax.axis_index("c")
    s = lax.axis_index("s")

    def zb(i, _):
        rows_v[i // 4, pl.ds((i % 4) * 16, 16)] = jnp.zeros((16,), _f32)
        return 0
    lax.fori_loop(0, CK * 4, zb, 0, unroll=8)
    for k in range(12):
        pltpu.sync_copy(rows_v, out_sp.at[pl.ds(s * TR + k * CK, CK), :])
    pltpu.sync_copy(rows_v.at[pl.ds(0, TR - 12 * CK), :],
                    out_sp.at[pl.ds(s * TR + 12 * CK, TR - 12 * CK), :])
    plsc.subcore_barrier()

    for t in range(2):
        w = s + t * 16

        def group(g, _):
            base = w * CH + g * GC
            pltpu.sync_copy(row_hbm.at[pl.ds(base, GC), :], rowv)
            pltpu.sync_copy(lc_hbm.at[pl.ds(c * (EP // CK) + base, GC), :], lcv)
            pltpu.sync_copy(norm_hbm.at[pl.ds(base, GC), :], normv)

            def chunk(j, _):
                pltpu.async_copy(x_hbm.at[rowv.at[j]], rows_v, sem).wait()

                def edge(k, _):
                    sc = plsc.load_gather(normv, [jnp.full((16,), j, _i32),
                                                  jnp.full((16,), k, _i32)])
                    for q in range(4):
                        rows_v[k, pl.ds(q * 16, 16)] = rows_v[k, pl.ds(q * 16, 16)] * sc
                    return 0
                lax.fori_loop(0, CK, edge, 0, unroll=4)
                pltpu.sync_copy(rows_v, out_sp.at[lcv.at[j]], add=True)
                return 0
            lax.fori_loop(0, GC, chunk, 0)
            return 0
        lax.fori_loop(0, CH // GC, group, 0)
    plsc.subcore_barrier()

    for k in range(13):
        cnt = CK if k < 12 else TR - 12 * CK
        off = s * TR + k * CK
        pltpu.sync_copy(out_sp.at[pl.ds(off, cnt), :], rows_v.at[pl.ds(0, cnt), :])

        def rl(i, _):
            r = i // 4
            q = (i % 4) * 16
            rows_v[r, pl.ds(q, 16)] = jnp.maximum(rows_v[r, pl.ds(q, 16)], 0.0)
            return 0
        lax.fori_loop(0, cnt * 4, rl, 0, unroll=4)
        pltpu.sync_copy(rows_v.at[pl.ds(0, cnt), :], out_hbm.at[c, pl.ds(off, cnt), :])


_layer_kernel = functools.partial(
    pl.kernel,
    out_type=jax.ShapeDtypeStruct((2, HP, DD), _f32),
    mesh=_mesh,
    compiler_params=pltpu.CompilerParams(needs_layout_passes=False,
                                         use_tc_tiling_on_sc=False),
    scratch_types=[
        pltpu.VMEM((GC, CK), _i32),
        pltpu.VMEM((GC, CK), _i32),
        pltpu.VMEM((GC, CK), _f32),
        pltpu.VMEM((CK, DD), _f32),
        pltpu.SemaphoreType.DMA,
        pltpu.VMEM_SHARED((HP, DD), _f32),
    ],
)(_layer_body)


def _final_body(x0, r1, r2, w_ref, b_ref, o):
    t = (x0[...] + r1[...] + r2[...]) * (1.0 / 3.0)
    o[...] = lax.dot_general(t, w_ref[...], (((1,), (1,)), ((), ())),
                             preferred_element_type=_f32) + b_ref[...]


def _final(x0, r1, r2, W, b):
    blk = 400
    grid = NN // blk
    return pl.pallas_call(
        _final_body,
        grid=(grid,),
        in_specs=[
            pl.BlockSpec((blk, DD), lambda i: (i, 0)),
            pl.BlockSpec((blk, DD), lambda i: (i, 0)),
            pl.BlockSpec((blk, DD), lambda i: (i, 0)),
            pl.BlockSpec((DD, DD), lambda i: (0, 0)),
            pl.BlockSpec((1, DD), lambda i: (0, 0)),
        ],
        out_specs=pl.BlockSpec((blk, DD), lambda i: (i, 0)),
        out_shape=jax.ShapeDtypeStruct((NN, DD), _f32),
    )(x0, r1, r2, W, b.reshape(1, DD))


@jax.jit
def kernel(edge_index, edge_weight, emb_users, emb_items, W, b):
    row = edge_index[0].astype(_i32)
    col = edge_index[1].astype(_i32)
    pad = EP - EE
    row3 = jnp.concatenate([row, jnp.zeros((pad,), _i32)]).reshape(NW, CH, CK)
    col3 = jnp.concatenate([col, jnp.full((pad,), NN, _i32)]).reshape(NW, CH, CK)
    ew3 = jnp.concatenate([edge_weight, jnp.zeros((pad,), _f32)]).reshape(NW, CH, CK)

    deg2 = _deg_kernel(col3, ew3).reshape(2, NP_)
    d0 = deg2[0].reshape(NP_ // 128, 128)
    d1 = deg2[1].reshape(NP_ // 128, 128)
    dis = pl.pallas_call(
        _dis_body,
        out_shape=jax.ShapeDtypeStruct((NP_ // 128, 128), _f32),
    )(d0, d1).reshape(NP_)

    row1 = row3.reshape(EP)
    col1 = col3.reshape(EP)
    ew1 = ew3.reshape(EP)
    norm1, lc1f = _norm_kernel(dis, row1, col1, ew1)
    norm3 = norm1.reshape(NW, CH, CK)
    lc4 = lc1f.reshape(2, NW, CH, CK)

    x0 = jnp.concatenate([emb_users, emb_items], axis=0)
    row2 = row1.reshape(EP // CK, CK)
    lc2 = lc1f.reshape(2 * EP // CK, CK)
    norm2 = norm1.reshape(EP // CK, CK)
    y1 = _layer_kernel(x0, row2, lc2, norm2)
    r1 = y1[:, :HALF, :].reshape(NN, DD)
    y2 = _layer_kernel(r1, row2, lc2, norm2)
    r2 = y2[:, :HALF, :].reshape(NN, DD)

    out = _final(x0, r1, r2, W, b)
    return (out[:NU], emb_users, out[NU:], emb_items)


# speedup vs baseline: 10.1250x; 1.2657x over previous
import functools
import jax
import jax.numpy as jnp
from jax import lax
from jax.experimental import pallas as pl
from jax.experimental.pallas import tpu as pltpu
from jax.experimental.pallas import tpu_sc as plsc

NU = 25000
NI = 25000
NN = NU + NI
EE = 800000
DD = 64
NW = 32
CK = 128
CH = 196
EW_ = CK * CH
EP = NW * EW_
NP_ = 50176
NT = 3136
HALF = 25000
HP = 25088
TR = 1568

_mesh = plsc.VectorSubcoreMesh(core_axis_name="c", subcore_axis_name="s")
_f32 = jnp.float32
_i32 = jnp.int32


def _deg_body(col_hbm, ew_hbm, out_hbm, colv, ewv, zbuf, deg_sp):
    c = lax.axis_index("c")
    s = lax.axis_index("s")
    w = s * 2 + c

    def zb(i, _):
        zbuf[pl.ds(i * 16, 16)] = jnp.zeros((16,), _f32)
        return 0
    lax.fori_loop(0, NT // 16, zb, 0, unroll=8)
    pltpu.sync_copy(zbuf, deg_sp.at[pl.ds(s * NT, NT)])
    plsc.subcore_barrier()

    pltpu.sync_copy(col_hbm.at[w], colv)
    pltpu.sync_copy(ew_hbm.at[w], ewv)

    def chunk(j, _):
        pltpu.sync_copy(ewv.at[j], deg_sp.at[colv.at[j]], add=True)
        return 0
    lax.fori_loop(0, CH, chunk, 0)
    plsc.subcore_barrier()
    pltpu.sync_copy(deg_sp.at[pl.ds(s * NT, NT)], zbuf)
    pltpu.sync_copy(zbuf, out_hbm.at[pl.ds(c * NP_ + s * NT, NT)])


_deg_kernel = functools.partial(
    pl.kernel,
    out_type=jax.ShapeDtypeStruct((2 * NP_,), _f32),
    mesh=_mesh,
    compiler_params=pltpu.CompilerParams(needs_layout_passes=False),
    scratch_types=[
        pltpu.VMEM((CH, CK), _i32),
        pltpu.VMEM((CH, CK), _f32),
        pltpu.VMEM((NT,), _f32),
        pltpu.VMEM_SHARED((NP_,), _f32),
    ],
)(_deg_body)


def _dis_body(d0, d1, o):
    d = d0[...] + d1[...]
    o[...] = jnp.where(d > 0.0, lax.rsqrt(d), 0.0)


QN = EW_ // 2


def _norm_body(dis_hbm, row_hbm, col_hbm, ew_hbm, norm_hbm, lc_hbm,
               disv, rowq, colq, ewq, normq, lc0q, lc1q):
    c = lax.axis_index("c")
    s = lax.axis_index("s")
    w = s * 2 + c
    pltpu.sync_copy(dis_hbm, disv)
    for h in range(2):
        off = w * EW_ + h * QN
        pltpu.sync_copy(row_hbm.at[pl.ds(off, QN)], rowq)
        pltpu.sync_copy(col_hbm.at[pl.ds(off, QN)], colq)
        pltpu.sync_copy(ew_hbm.at[pl.ds(off, QN)], ewq)

        def step(t, _):
            i = t * 16
            rv = rowq[pl.ds(i, 16)]
            cv = colq[pl.ds(i, 16)]
            ev = ewq[pl.ds(i, 16)]
            a = plsc.load_gather(disv, [rv])
            b = plsc.load_gather(disv, [cv])
            normq[pl.ds(i, 16)] = a * b * ev
            dummy = jnp.full((16,), HALF, _i32)
            lc0q[pl.ds(i, 16)] = jnp.where(cv < HALF, cv, dummy)
            l1 = cv - HALF
            ok1 = (l1 >= 0) & (l1 < HALF)
            lc1q[pl.ds(i, 16)] = jnp.where(ok1, l1, dummy)
            return 0
        lax.fori_loop(0, QN // 16, step, 0, unroll=2)
        pltpu.sync_copy(normq, norm_hbm.at[pl.ds(off, QN)])
        pltpu.sync_copy(lc0q, lc_hbm.at[pl.ds(off, QN)])
        pltpu.sync_copy(lc1q, lc_hbm.at[pl.ds(EP + off, QN)])


_norm_kernel = functools.partial(
    pl.kernel,
    out_type=(jax.ShapeDtypeStruct((EP,), _f32),
              jax.ShapeDtypeStruct((2 * EP,), _i32)),
    mesh=_mesh,
    compiler_params=pltpu.CompilerParams(needs_layout_passes=False),
    scratch_types=[
        pltpu.VMEM((NP_,), _f32),
        pltpu.VMEM((QN,), _i32),
        pltpu.VMEM((QN,), _i32),
        pltpu.VMEM((QN,), _f32),
        pltpu.VMEM((QN,), _f32),
        pltpu.VMEM((QN,), _i32),
        pltpu.VMEM((QN,), _i32),
    ],
)(_norm_body)


GC = 7


def _layer_body(x_hbm, row_hbm, lc_hbm, norm_hbm, out_hbm,
                rowv, lcv, normv, rows_a, rows_b, sga, sgb, out_sp):
    c = lax.axis_index("c")
    s = lax.axis_index("s")

    def zb(i, _):
        rows_a[i // 4, pl.ds((i % 4) * 16, 16)] = jnp.zeros((16,), _f32)
        return 0
    lax.fori_loop(0, CK * 4, zb, 0, unroll=8)
    for k in range(12):
        pltpu.sync_copy(rows_a, out_sp.at[pl.ds(s * TR + k * CK, CK), :])
    pltpu.sync_copy(rows_a.at[pl.ds(0, TR - 12 * CK), :],
                    out_sp.at[pl.ds(s * TR + 12 * CK, TR - 12 * CK), :])
    plsc.subcore_barrier()

    bufs = (rows_a, rows_b)
    sems = (sga, sgb)
    for t in range(2):
        w = s + t * 16

        def group(g, _):
            base = w * CH + g * GC
            pltpu.sync_copy(row_hbm.at[pl.ds(base, GC), :], rowv)
            pltpu.sync_copy(lc_hbm.at[pl.ds(c * (EP // CK) + base, GC), :], lcv)
            pltpu.sync_copy(norm_hbm.at[pl.ds(base, GC), :], normv)

            cps = [pltpu.async_copy(x_hbm.at[rowv.at[0]], rows_a, sga), None]
            for j in range(GC):
                b = j % 2
                if j + 1 < GC:
                    cps[1 - b] = pltpu.async_copy(
                        x_hbm.at[rowv.at[j + 1]], bufs[1 - b], sems[1 - b])
                cps[b].wait()
                buf = bufs[b]

                def edge(k, _):
                    sc = plsc.load_gather(normv, [jnp.full((16,), j, _i32),
                                                  jnp.full((16,), k, _i32)])
                    for q in range(4):
                        buf[k, pl.ds(q * 16, 16)] = buf[k, pl.ds(q * 16, 16)] * sc
                    return 0
                lax.fori_loop(0, CK, edge, 0, unroll=4)
                pltpu.sync_copy(buf, out_sp.at[lcv.at[j]], add=True)
            return 0
        lax.fori_loop(0, CH // GC, group, 0)
    plsc.subcore_barrier()

    for k in range(13):
        cnt = CK if k < 12 else TR - 12 * CK
        off = s * TR + k * CK
        pltpu.sync_copy(out_sp.at[pl.ds(off, cnt), :], rows_a.at[pl.ds(0, cnt), :])

        def rl(i, _):
            r = i // 4
            q = (i % 4) * 16
            rows_a[r, pl.ds(q, 16)] = jnp.maximum(rows_a[r, pl.ds(q, 16)], 0.0)
            return 0
        lax.fori_loop(0, cnt * 4, rl, 0, unroll=4)
        pltpu.sync_copy(rows_a.at[pl.ds(0, cnt), :], out_hbm.at[c, pl.ds(off, cnt), :])


_layer_kernel = functools.partial(
    pl.kernel,
    out_type=jax.ShapeDtypeStruct((2, HP, DD), _f32),
    mesh=_mesh,
    compiler_params=pltpu.CompilerParams(needs_layout_passes=False,
                                         use_tc_tiling_on_sc=False),
    scratch_types=[
        pltpu.VMEM((GC, CK), _i32),
        pltpu.VMEM((GC, CK), _i32),
        pltpu.VMEM((GC, CK), _f32),
        pltpu.VMEM((CK, DD), _f32),
        pltpu.VMEM((CK, DD), _f32),
        pltpu.SemaphoreType.DMA,
        pltpu.SemaphoreType.DMA,
        pltpu.VMEM_SHARED((HP, DD), _f32),
    ],
)(_layer_body)


def _final_body(x0, r1, r2, w_ref, b_ref, o):
    t = (x0[...] + r1[...] + r2[...]) * (1.0 / 3.0)
    o[...] = lax.dot_general(t, w_ref[...], (((1,), (1,)), ((), ())),
                             preferred_element_type=_f32) + b_ref[...]


def _final(x0, r1, r2, W, b):
    blk = 400
    grid = NN // blk
    return pl.pallas_call(
        _final_body,
        grid=(grid,),
        in_specs=[
            pl.BlockSpec((blk, DD), lambda i: (i, 0)),
            pl.BlockSpec((blk, DD), lambda i: (i, 0)),
            pl.BlockSpec((blk, DD), lambda i: (i, 0)),
            pl.BlockSpec((DD, DD), lambda i: (0, 0)),
            pl.BlockSpec((1, DD), lambda i: (0, 0)),
        ],
        out_specs=pl.BlockSpec((blk, DD), lambda i: (i, 0)),
        out_shape=jax.ShapeDtypeStruct((NN, DD), _f32),
    )(x0, r1, r2, W, b.reshape(1, DD))


@jax.jit
def kernel(edge_index, edge_weight, emb_users, emb_items, W, b):
    row = edge_index[0].astype(_i32)
    col = edge_index[1].astype(_i32)
    pad = EP - EE
    row3 = jnp.concatenate([row, jnp.zeros((pad,), _i32)]).reshape(NW, CH, CK)
    col3 = jnp.concatenate([col, jnp.full((pad,), NN, _i32)]).reshape(NW, CH, CK)
    ew3 = jnp.concatenate([edge_weight, jnp.zeros((pad,), _f32)]).reshape(NW, CH, CK)

    deg2 = _deg_kernel(col3, ew3).reshape(2, NP_)
    d0 = deg2[0].reshape(NP_ // 128, 128)
    d1 = deg2[1].reshape(NP_ // 128, 128)
    dis = pl.pallas_call(
        _dis_body,
        out_shape=jax.ShapeDtypeStruct((NP_ // 128, 128), _f32),
    )(d0, d1).reshape(NP_)

    row1 = row3.reshape(EP)
    col1 = col3.reshape(EP)
    ew1 = ew3.reshape(EP)
    norm1, lc1f = _norm_kernel(dis, row1, col1, ew1)

    x0 = jnp.concatenate([emb_users, emb_items], axis=0)
    row2 = row1.reshape(EP // CK, CK)
    lc2 = lc1f.reshape(2 * EP // CK, CK)
    norm2 = norm1.reshape(EP // CK, CK)
    y1 = _layer_kernel(x0, row2, lc2, norm2)
    r1 = y1[:, :HALF, :].reshape(NN, DD)
    y2 = _layer_kernel(r1, row2, lc2, norm2)
    r2 = y2[:, :HALF, :].reshape(NN, DD)

    out = _final(x0, r1, r2, W, b)
    return (out[:NU], emb_users, out[NU:], emb_items)


# speedup vs baseline: 11.4119x; 1.1271x over previous
import functools
import jax
import jax.numpy as jnp
from jax import lax
from jax.experimental import pallas as pl
from jax.experimental.pallas import tpu as pltpu
from jax.experimental.pallas import tpu_sc as plsc

NU = 25000
NI = 25000
NN = NU + NI
EE = 800000
DD = 64
NW = 32
CK = 128
CH = 196
EW_ = CK * CH
EP = NW * EW_
NP_ = 50176
NT = 3136
HALF = 25000
HP = 25088
TR = 1568

_mesh = plsc.VectorSubcoreMesh(core_axis_name="c", subcore_axis_name="s")
_f32 = jnp.float32
_i32 = jnp.int32


def _deg_body(col_hbm, ew_hbm, out_hbm, colv, ewv, zbuf, deg_sp):
    c = lax.axis_index("c")
    s = lax.axis_index("s")
    w = s * 2 + c

    def zb(i, _):
        zbuf[pl.ds(i * 16, 16)] = jnp.zeros((16,), _f32)
        return 0
    lax.fori_loop(0, NT // 16, zb, 0, unroll=8)
    pltpu.sync_copy(zbuf, deg_sp.at[pl.ds(s * NT, NT)])
    plsc.subcore_barrier()

    pltpu.sync_copy(col_hbm.at[w], colv)
    pltpu.sync_copy(ew_hbm.at[w], ewv)

    def chunk(j, _):
        pltpu.sync_copy(ewv.at[j], deg_sp.at[colv.at[j]], add=True)
        return 0
    lax.fori_loop(0, CH, chunk, 0)
    plsc.subcore_barrier()
    pltpu.sync_copy(deg_sp.at[pl.ds(s * NT, NT)], zbuf)
    pltpu.sync_copy(zbuf, out_hbm.at[pl.ds(c * NP_ + s * NT, NT)])


_deg_kernel = functools.partial(
    pl.kernel,
    out_type=jax.ShapeDtypeStruct((2 * NP_,), _f32),
    mesh=_mesh,
    compiler_params=pltpu.CompilerParams(needs_layout_passes=False),
    scratch_types=[
        pltpu.VMEM((CH, CK), _i32),
        pltpu.VMEM((CH, CK), _f32),
        pltpu.VMEM((NT,), _f32),
        pltpu.VMEM_SHARED((NP_,), _f32),
    ],
)(_deg_body)


def _dis_body(d0, d1, o):
    d = d0[...] + d1[...]
    o[...] = jnp.where(d > 0.0, lax.rsqrt(d), 0.0)


QN = EW_ // 2


def _norm_body(dis_hbm, row_hbm, col_hbm, ew_hbm, norm_hbm, lc_hbm,
               disv, rowq, colq, ewq, normq, lc0q, lc1q):
    c = lax.axis_index("c")
    s = lax.axis_index("s")
    w = s * 2 + c
    pltpu.sync_copy(dis_hbm, disv)
    for h in range(2):
        off = w * EW_ + h * QN
        pltpu.sync_copy(row_hbm.at[pl.ds(off, QN)], rowq)
        pltpu.sync_copy(col_hbm.at[pl.ds(off, QN)], colq)
        pltpu.sync_copy(ew_hbm.at[pl.ds(off, QN)], ewq)

        def step(t, _):
            i = t * 16
            rv = rowq[pl.ds(i, 16)]
            cv = colq[pl.ds(i, 16)]
            ev = ewq[pl.ds(i, 16)]
            a = plsc.load_gather(disv, [rv])
            b = plsc.load_gather(disv, [cv])
            normq[pl.ds(i, 16)] = a * b * ev
            dummy = jnp.full((16,), HALF, _i32)
            lc0q[pl.ds(i, 16)] = jnp.where(cv < HALF, cv, dummy)
            l1 = cv - HALF
            ok1 = (l1 >= 0) & (l1 < HALF)
            lc1q[pl.ds(i, 16)] = jnp.where(ok1, l1, dummy)
            return 0
        lax.fori_loop(0, QN // 16, step, 0, unroll=2)
        pltpu.sync_copy(normq, norm_hbm.at[pl.ds(off, QN)])
        pltpu.sync_copy(lc0q, lc_hbm.at[pl.ds(off, QN)])
        pltpu.sync_copy(lc1q, lc_hbm.at[pl.ds(EP + off, QN)])


_norm_kernel = functools.partial(
    pl.kernel,
    out_type=(jax.ShapeDtypeStruct((EP,), _f32),
              jax.ShapeDtypeStruct((2 * EP,), _i32)),
    mesh=_mesh,
    compiler_params=pltpu.CompilerParams(needs_layout_passes=False),
    scratch_types=[
        pltpu.VMEM((NP_,), _f32),
        pltpu.VMEM((QN,), _i32),
        pltpu.VMEM((QN,), _i32),
        pltpu.VMEM((QN,), _f32),
        pltpu.VMEM((QN,), _f32),
        pltpu.VMEM((QN,), _i32),
        pltpu.VMEM((QN,), _i32),
    ],
)(_norm_body)


GC = 7


def _layer_body(x_hbm, row_hbm, lc_hbm, norm_hbm, out_hbm,
                rowv, lcv, normv, rows_a, rows_b, sga, sgb, ssa, ssb, out_sp):
    c = lax.axis_index("c")
    s = lax.axis_index("s")

    def zb(i, _):
        rows_a[i // 4, pl.ds((i % 4) * 16, 16)] = jnp.zeros((16,), _f32)
        return 0
    lax.fori_loop(0, CK * 4, zb, 0, unroll=8)
    for k in range(12):
        pltpu.sync_copy(rows_a, out_sp.at[pl.ds(s * TR + k * CK, CK), :])
    pltpu.sync_copy(rows_a.at[pl.ds(0, TR - 12 * CK), :],
                    out_sp.at[pl.ds(s * TR + 12 * CK, TR - 12 * CK), :])
    plsc.subcore_barrier()

    bufs = (rows_a, rows_b)
    sems = (sga, sgb)
    ssems = (ssa, ssb)
    for t in range(2):
        w = s + t * 16

        def group(g, _):
            base = w * CH + g * GC
            pltpu.sync_copy(row_hbm.at[pl.ds(base, GC), :], rowv)
            pltpu.sync_copy(lc_hbm.at[pl.ds(c * (EP // CK) + base, GC), :], lcv)
            pltpu.sync_copy(norm_hbm.at[pl.ds(base, GC), :], normv)

            gcp = [pltpu.async_copy(x_hbm.at[rowv.at[0]], rows_a, sga), None]
            scp = [None, None]
            for j in range(GC):
                b = j % 2
                if j + 1 < GC:
                    ob = 1 - b
                    if scp[ob] is not None:
                        scp[ob].wait()
                        scp[ob] = None
                    gcp[ob] = pltpu.async_copy(
                        x_hbm.at[rowv.at[j + 1]], bufs[ob], sems[ob])
                gcp[b].wait()
                buf = bufs[b]
                jj = j

                @plsc.parallel_loop(0, CK, unroll=4)
                def edge(k, _buf=buf, _j=jj):
                    sc = plsc.load_gather(normv, [jnp.full((16,), _j, _i32),
                                                  jnp.full((16,), k, _i32)])
                    for q in range(4):
                        _buf[k, pl.ds(q * 16, 16)] = _buf[k, pl.ds(q * 16, 16)] * sc
                if j < GC - 1:
                    scp[b] = pltpu.async_copy(buf, out_sp.at[lcv.at[j]],
                                              ssems[b], add=True)
                else:
                    pltpu.sync_copy(buf, out_sp.at[lcv.at[j]], add=True)
            for x in (0, 1):
                if scp[x] is not None:
                    scp[x].wait()
            return 0
        lax.fori_loop(0, CH // GC, group, 0)
    plsc.subcore_barrier()

    for k in range(13):
        cnt = CK if k < 12 else TR - 12 * CK
        off = s * TR + k * CK
        pltpu.sync_copy(out_sp.at[pl.ds(off, cnt), :], rows_a.at[pl.ds(0, cnt), :])

        def rl(i, _):
            r = i // 4
            q = (i % 4) * 16
            rows_a[r, pl.ds(q, 16)] = jnp.maximum(rows_a[r, pl.ds(q, 16)], 0.0)
            return 0
        lax.fori_loop(0, cnt * 4, rl, 0, unroll=4)
        pltpu.sync_copy(rows_a.at[pl.ds(0, cnt), :], out_hbm.at[c, pl.ds(off, cnt), :])


_layer_kernel = functools.partial(
    pl.kernel,
    out_type=jax.ShapeDtypeStruct((2, HP, DD), _f32),
    mesh=_mesh,
    compiler_params=pltpu.CompilerParams(needs_layout_passes=False,
                                         use_tc_tiling_on_sc=False),
    scratch_types=[
        pltpu.VMEM((GC, CK), _i32),
        pltpu.VMEM((GC, CK), _i32),
        pltpu.VMEM((GC, CK), _f32),
        pltpu.VMEM((CK, DD), _f32),
        pltpu.VMEM((CK, DD), _f32),
        pltpu.SemaphoreType.DMA,
        pltpu.SemaphoreType.DMA,
        pltpu.SemaphoreType.DMA,
        pltpu.SemaphoreType.DMA,
        pltpu.VMEM_SHARED((HP, DD), _f32),
    ],
)(_layer_body)


def _final_body(x0, r1, r2, w_ref, b_ref, o):
    t = (x0[...] + r1[...] + r2[...]) * (1.0 / 3.0)
    o[...] = lax.dot_general(t, w_ref[...], (((1,), (1,)), ((), ())),
                             preferred_element_type=_f32) + b_ref[...]


def _final(x0, r1, r2, W, b):
    blk = 400
    grid = NN // blk
    return pl.pallas_call(
        _final_body,
        grid=(grid,),
        in_specs=[
            pl.BlockSpec((blk, DD), lambda i: (i, 0)),
            pl.BlockSpec((blk, DD), lambda i: (i, 0)),
            pl.BlockSpec((blk, DD), lambda i: (i, 0)),
            pl.BlockSpec((DD, DD), lambda i: (0, 0)),
            pl.BlockSpec((1, DD), lambda i: (0, 0)),
        ],
        out_specs=pl.BlockSpec((blk, DD), lambda i: (i, 0)),
        out_shape=jax.ShapeDtypeStruct((NN, DD), _f32),
    )(x0, r1, r2, W, b.reshape(1, DD))


@jax.jit
def kernel(edge_index, edge_weight, emb_users, emb_items, W, b):
    row = edge_index[0].astype(_i32)
    col = edge_index[1].astype(_i32)
    pad = EP - EE
    row3 = jnp.concatenate([row, jnp.zeros((pad,), _i32)]).reshape(NW, CH, CK)
    col3 = jnp.concatenate([col, jnp.full((pad,), NN, _i32)]).reshape(NW, CH, CK)
    ew3 = jnp.concatenate([edge_weight, jnp.zeros((pad,), _f32)]).reshape(NW, CH, CK)

    deg2 = _deg_kernel(col3, ew3).reshape(2, NP_)
    d0 = deg2[0].reshape(NP_ // 128, 128)
    d1 = deg2[1].reshape(NP_ // 128, 128)
    dis = pl.pallas_call(
        _dis_body,
        out_shape=jax.ShapeDtypeStruct((NP_ // 128, 128), _f32),
    )(d0, d1).reshape(NP_)

    row1 = row3.reshape(EP)
    col1 = col3.reshape(EP)
    ew1 = ew3.reshape(EP)
    norm1, lc1f = _norm_kernel(dis, row1, col1, ew1)

    x0 = jnp.concatenate([emb_users, emb_items], axis=0)
    row2 = row1.reshape(EP // CK, CK)
    lc2 = lc1f.reshape(2 * EP // CK, CK)
    norm2 = norm1.reshape(EP // CK, CK)
    y1 = _layer_kernel(x0, row2, lc2, norm2)
    r1 = y1[:, :HALF, :].reshape(NN, DD)
    y2 = _layer_kernel(r1, row2, lc2, norm2)
    r2 = y2[:, :HALF, :].reshape(NN, DD)

    out = _final(x0, r1, r2, W, b)
    return (out[:NU], emb_users, out[NU:], emb_items)


# speedup vs baseline: 14.5124x; 1.2717x over previous
import functools
import jax
import jax.numpy as jnp
from jax import lax
from jax.experimental import pallas as pl
from jax.experimental.pallas import tpu as pltpu
from jax.experimental.pallas import tpu_sc as plsc

NU = 25000
NI = 25000
NN = NU + NI
EE = 800000
DD = 64
NW = 32
CK = 128
CH = 196
EW_ = CK * CH
EP = NW * EW_
NP_ = 50176
NT = 3136
DW = 32
NHS = 50048
TR = NHS // 16

_mesh = plsc.VectorSubcoreMesh(core_axis_name="c", subcore_axis_name="s")
_f32 = jnp.float32
_i32 = jnp.int32


def _deg_body(col_hbm, ew_hbm, out_hbm, colv, ewv, zbuf, deg_sp):
    c = lax.axis_index("c")
    s = lax.axis_index("s")
    w = s * 2 + c

    def zb(i, _):
        zbuf[pl.ds(i * 16, 16)] = jnp.zeros((16,), _f32)
        return 0
    lax.fori_loop(0, NT // 16, zb, 0, unroll=8)
    pltpu.sync_copy(zbuf, deg_sp.at[pl.ds(s * NT, NT)])
    plsc.subcore_barrier()

    pltpu.sync_copy(col_hbm.at[w], colv)
    pltpu.sync_copy(ew_hbm.at[w], ewv)

    def chunk(j, _):
        pltpu.sync_copy(ewv.at[j], deg_sp.at[colv.at[j]], add=True)
        return 0
    lax.fori_loop(0, CH, chunk, 0)
    plsc.subcore_barrier()
    pltpu.sync_copy(deg_sp.at[pl.ds(s * NT, NT)], zbuf)
    pltpu.sync_copy(zbuf, out_hbm.at[pl.ds(c * NP_ + s * NT, NT)])


_deg_kernel = functools.partial(
    pl.kernel,
    out_type=jax.ShapeDtypeStruct((2 * NP_,), _f32),
    mesh=_mesh,
    compiler_params=pltpu.CompilerParams(needs_layout_passes=False),
    scratch_types=[
        pltpu.VMEM((CH, CK), _i32),
        pltpu.VMEM((CH, CK), _f32),
        pltpu.VMEM((NT,), _f32),
        pltpu.VMEM_SHARED((NP_,), _f32),
    ],
)(_deg_body)


def _dis_body(d0, d1, o):
    d = d0[...] + d1[...]
    o[...] = jnp.where(d > 0.0, lax.rsqrt(d), 0.0)


QN = EW_ // 2


def _norm_body(dis_hbm, row_hbm, col_hbm, ew_hbm, norm_hbm, lc_hbm,
               disv, rowq, colq, ewq, normq, lc0q, lc1q):
    c = lax.axis_index("c")
    s = lax.axis_index("s")
    w = s * 2 + c
    pltpu.sync_copy(dis_hbm, disv)
    for h in range(2):
        off = w * EW_ + h * QN
        pltpu.sync_copy(row_hbm.at[pl.ds(off, QN)], rowq)
        pltpu.sync_copy(col_hbm.at[pl.ds(off, QN)], colq)
        pltpu.sync_copy(ew_hbm.at[pl.ds(off, QN)], ewq)

        def step(t, _):
            i = t * 16
            rv = rowq[pl.ds(i, 16)]
            cv = colq[pl.ds(i, 16)]
            ev = ewq[pl.ds(i, 16)]
            a = plsc.load_gather(disv, [rv])
            b = plsc.load_gather(disv, [cv])
            normq[pl.ds(i, 16)] = a * b * ev
            lc0q[pl.ds(i, 16)] = rv
            lc1q[pl.ds(i, 16)] = rv + NHS
            return 0
        lax.fori_loop(0, QN // 16, step, 0, unroll=2)
        pltpu.sync_copy(normq, norm_hbm.at[pl.ds(off, QN)])
        pltpu.sync_copy(lc0q, lc_hbm.at[pl.ds(off, QN)])
        pltpu.sync_copy(lc1q, lc_hbm.at[pl.ds(EP + off, QN)])


_norm_kernel = functools.partial(
    pl.kernel,
    out_type=(jax.ShapeDtypeStruct((EP,), _f32),
              jax.ShapeDtypeStruct((2 * EP,), _i32)),
    mesh=_mesh,
    compiler_params=pltpu.CompilerParams(needs_layout_passes=False),
    scratch_types=[
        pltpu.VMEM((NP_,), _f32),
        pltpu.VMEM((QN,), _i32),
        pltpu.VMEM((QN,), _i32),
        pltpu.VMEM((QN,), _f32),
        pltpu.VMEM((QN,), _f32),
        pltpu.VMEM((QN,), _i32),
        pltpu.VMEM((QN,), _i32),
    ],
)(_norm_body)


GC = 7


def _layer_body(xs_hbm, ro_hbm, col_hbm, norm_hbm, out_hbm,
                rowv, colv, normv, rows_a, rows_b, sga, sgb, ssa, ssb, out_sp):
    c = lax.axis_index("c")
    s = lax.axis_index("s")
    nfull = TR // CK
    rem = TR - nfull * CK

    def zb(i, _):
        rows_a[i // 2, pl.ds((i % 2) * 16, 16)] = jnp.zeros((16,), _f32)
        return 0
    lax.fori_loop(0, CK * 2, zb, 0, unroll=8)
    for k in range(nfull):
        pltpu.sync_copy(rows_a, out_sp.at[pl.ds(s * TR + k * CK, CK), :])
    pltpu.sync_copy(rows_a.at[pl.ds(0, rem), :],
                    out_sp.at[pl.ds(s * TR + nfull * CK, rem), :])
    plsc.subcore_barrier()

    bufs = (rows_a, rows_b)
    sems = (sga, sgb)
    ssems = (ssa, ssb)
    for t in range(2):
        w = s + t * 16

        def group(g, _):
            base = w * CH + g * GC
            pltpu.sync_copy(ro_hbm.at[pl.ds(c * (EP // CK) + base, GC), :], rowv)
            pltpu.sync_copy(col_hbm.at[pl.ds(base, GC), :], colv)
            pltpu.sync_copy(norm_hbm.at[pl.ds(base, GC), :], normv)

            gcp = [pltpu.async_copy(xs_hbm.at[rowv.at[0]], rows_a, sga), None]
            scp = [None, None]
            for j in range(GC):
                b = j % 2
                if j + 1 < GC:
                    ob = 1 - b
                    if scp[ob] is not None:
                        scp[ob].wait()
                        scp[ob] = None
                    gcp[ob] = pltpu.async_copy(
                        xs_hbm.at[rowv.at[j + 1]], bufs[ob], sems[ob])
                gcp[b].wait()
                buf = bufs[b]
                jj = j

                @plsc.parallel_loop(0, CK, unroll=4)
                def edge(k, _buf=buf, _j=jj):
                    sc = plsc.load_gather(normv, [jnp.full((16,), _j, _i32),
                                                  jnp.full((16,), k, _i32)])
                    for q in range(2):
                        _buf[k, pl.ds(q * 16, 16)] = _buf[k, pl.ds(q * 16, 16)] * sc
                if j < GC - 1:
                    scp[b] = pltpu.async_copy(buf, out_sp.at[colv.at[j]],
                                              ssems[b], add=True)
                else:
                    pltpu.sync_copy(buf, out_sp.at[colv.at[j]], add=True)
            for x in (0, 1):
                if scp[x] is not None:
                    scp[x].wait()
            return 0
        lax.fori_loop(0, CH // GC, group, 0)
    plsc.subcore_barrier()

    for k in range(nfull + 1):
        cnt = CK if k < nfull else rem
        off = s * TR + k * CK
        pltpu.sync_copy(out_sp.at[pl.ds(off, cnt), :], rows_a.at[pl.ds(0, cnt), :])

        def rl(i, _):
            r = i // 2
            q = (i % 2) * 16
            rows_a[r, pl.ds(q, 16)] = jnp.maximum(rows_a[r, pl.ds(q, 16)], 0.0)
            return 0
        lax.fori_loop(0, cnt * 2, rl, 0, unroll=4)
        pltpu.sync_copy(rows_a.at[pl.ds(0, cnt), :], out_hbm.at[c, pl.ds(off, cnt), :])


_layer_kernel = functools.partial(
    pl.kernel,
    out_type=jax.ShapeDtypeStruct((2, NHS, DW), _f32),
    mesh=_mesh,
    compiler_params=pltpu.CompilerParams(needs_layout_passes=False,
                                         use_tc_tiling_on_sc=False),
    scratch_types=[
        pltpu.VMEM((GC, CK), _i32),
        pltpu.VMEM((GC, CK), _i32),
        pltpu.VMEM((GC, CK), _f32),
        pltpu.VMEM((CK, DW), _f32),
        pltpu.VMEM((CK, DW), _f32),
        pltpu.SemaphoreType.DMA,
        pltpu.SemaphoreType.DMA,
        pltpu.SemaphoreType.DMA,
        pltpu.SemaphoreType.DMA,
        pltpu.VMEM_SHARED((NHS, DW), _f32),
    ],
)(_layer_body)


def _final_body(x0, r1, r2, w_ref, b_ref, o):
    t = (x0[...] + r1[...] + r2[...]) * (1.0 / 3.0)
    o[...] = lax.dot_general(t, w_ref[...], (((1,), (1,)), ((), ())),
                             preferred_element_type=_f32) + b_ref[...]


def _final(x0, r1, r2, W, b):
    blk = 400
    grid = NN // blk
    return pl.pallas_call(
        _final_body,
        grid=(grid,),
        in_specs=[
            pl.BlockSpec((blk, DD), lambda i: (i, 0)),
            pl.BlockSpec((blk, DD), lambda i: (i, 0)),
            pl.BlockSpec((blk, DD), lambda i: (i, 0)),
            pl.BlockSpec((DD, DD), lambda i: (0, 0)),
            pl.BlockSpec((1, DD), lambda i: (0, 0)),
        ],
        out_specs=pl.BlockSpec((blk, DD), lambda i: (i, 0)),
        out_shape=jax.ShapeDtypeStruct((NN, DD), _f32),
    )(x0, r1, r2, W, b.reshape(1, DD))


@jax.jit
def kernel(edge_index, edge_weight, emb_users, emb_items, W, b):
    row = edge_index[0].astype(_i32)
    col = edge_index[1].astype(_i32)
    pad = EP - EE
    row3 = jnp.concatenate([row, jnp.zeros((pad,), _i32)]).reshape(NW, CH, CK)
    col3 = jnp.concatenate([col, jnp.full((pad,), NN, _i32)]).reshape(NW, CH, CK)
    ew3 = jnp.concatenate([edge_weight, jnp.zeros((pad,), _f32)]).reshape(NW, CH, CK)

    deg2 = _deg_kernel(col3, ew3).reshape(2, NP_)
    d0 = deg2[0].reshape(NP_ // 128, 128)
    d1 = deg2[1].reshape(NP_ // 128, 128)
    dis = pl.pallas_call(
        _dis_body,
        out_shape=jax.ShapeDtypeStruct((NP_ // 128, 128), _f32),
    )(d0, d1).reshape(NP_)

    row1 = row3.reshape(EP)
    col1 = col3.reshape(EP)
    ew1 = ew3.reshape(EP)
    norm1, lc1f = _norm_kernel(dis, row1, col1, ew1)

    x0 = jnp.concatenate([emb_users, emb_items], axis=0)
    padr = jnp.zeros((NHS - NN, DW), _f32)
    xs0 = jnp.concatenate([x0[:, :DW], padr, x0[:, DW:], padr], axis=0)
    col2 = col1.reshape(EP // CK, CK)
    ro2 = lc1f.reshape(2 * EP // CK, CK)
    norm2 = norm1.reshape(EP // CK, CK)
    y1 = _layer_kernel(xs0, ro2, col2, norm2)
    y2 = _layer_kernel(y1.reshape(2 * NHS, DW), ro2, col2, norm2)
    r1 = jnp.concatenate([y1[0, :NN], y1[1, :NN]], axis=1)
    r2 = jnp.concatenate([y2[0, :NN], y2[1, :NN]], axis=1)

    out = _final(x0, r1, r2, W, b)
    return (out[:NU], emb_users, out[NU:], emb_items)


# speedup vs baseline: 19.8592x; 1.3684x over previous
import functools
import jax
import jax.numpy as jnp
from jax import lax
from jax.experimental import pallas as pl
from jax.experimental.pallas import tpu as pltpu
from jax.experimental.pallas import tpu_sc as plsc

NU = 25000
NI = 25000
NN = NU + NI
EE = 800000
DD = 64
NW = 32
CK = 128
CH = 196
EW_ = CK * CH
EP = NW * EW_
NP_ = 50176
NT = 3136
DW = 32
NHS = 50048
TR = NHS // 16

_mesh = plsc.VectorSubcoreMesh(core_axis_name="c", subcore_axis_name="s")
_f32 = jnp.float32
_i32 = jnp.int32


def _deg_body(col_hbm, ew_hbm, out_hbm, colv, ewv, zbuf, deg_sp):
    c = lax.axis_index("c")
    s = lax.axis_index("s")
    w = s * 2 + c

    def zb(i, _):
        zbuf[pl.ds(i * 16, 16)] = jnp.zeros((16,), _f32)
        return 0
    lax.fori_loop(0, NT // 16, zb, 0, unroll=8)
    pltpu.sync_copy(zbuf, deg_sp.at[pl.ds(s * NT, NT)])
    plsc.subcore_barrier()

    pltpu.sync_copy(col_hbm.at[w], colv)
    pltpu.sync_copy(ew_hbm.at[w], ewv)

    def chunk(j, _):
        pltpu.sync_copy(ewv.at[j], deg_sp.at[colv.at[j]], add=True)
        return 0
    lax.fori_loop(0, CH, chunk, 0)
    plsc.subcore_barrier()
    pltpu.sync_copy(deg_sp.at[pl.ds(s * NT, NT)], zbuf)
    pltpu.sync_copy(zbuf, out_hbm.at[pl.ds(c * NP_ + s * NT, NT)])


_deg_kernel = functools.partial(
    pl.kernel,
    out_type=jax.ShapeDtypeStruct((2 * NP_,), _f32),
    mesh=_mesh,
    compiler_params=pltpu.CompilerParams(needs_layout_passes=False),
    scratch_types=[
        pltpu.VMEM((CH, CK), _i32),
        pltpu.VMEM((CH, CK), _f32),
        pltpu.VMEM((NT,), _f32),
        pltpu.VMEM_SHARED((NP_,), _f32),
    ],
)(_deg_body)


def _dis_body(d0, d1, o):
    d = d0[...] + d1[...]
    o[...] = jnp.where(d > 0.0, lax.rsqrt(d), 0.0)


QN = EW_ // 2


def _norm_body(dis_hbm, row_hbm, col_hbm, ew_hbm, norm_hbm, lc_hbm,
               disv, rowq, colq, ewq, normq, lc0q, lc1q):
    c = lax.axis_index("c")
    s = lax.axis_index("s")
    w = s * 2 + c
    pltpu.sync_copy(dis_hbm, disv)
    for h in range(2):
        off = w * EW_ + h * QN
        pltpu.sync_copy(row_hbm.at[pl.ds(off, QN)], rowq)
        pltpu.sync_copy(col_hbm.at[pl.ds(off, QN)], colq)
        pltpu.sync_copy(ew_hbm.at[pl.ds(off, QN)], ewq)

        def step(t, _):
            i = t * 16
            rv = rowq[pl.ds(i, 16)]
            cv = colq[pl.ds(i, 16)]
            ev = ewq[pl.ds(i, 16)]
            a = plsc.load_gather(disv, [rv])
            b = plsc.load_gather(disv, [cv])
            normq[pl.ds(i, 16)] = a * b * ev
            lc0q[pl.ds(i, 16)] = rv
            lc1q[pl.ds(i, 16)] = rv + NHS
            return 0
        lax.fori_loop(0, QN // 16, step, 0, unroll=2)
        pltpu.sync_copy(normq, norm_hbm.at[pl.ds(off, QN)])
        pltpu.sync_copy(lc0q, lc_hbm.at[pl.ds(off, QN)])
        pltpu.sync_copy(lc1q, lc_hbm.at[pl.ds(EP + off, QN)])


_norm_kernel = functools.partial(
    pl.kernel,
    out_type=(jax.ShapeDtypeStruct((EP,), _f32),
              jax.ShapeDtypeStruct((2 * EP,), _i32)),
    mesh=_mesh,
    compiler_params=pltpu.CompilerParams(needs_layout_passes=False),
    scratch_types=[
        pltpu.VMEM((NP_,), _f32),
        pltpu.VMEM((QN,), _i32),
        pltpu.VMEM((QN,), _i32),
        pltpu.VMEM((QN,), _f32),
        pltpu.VMEM((QN,), _f32),
        pltpu.VMEM((QN,), _i32),
        pltpu.VMEM((QN,), _i32),
    ],
)(_norm_body)


GC = 14
NB = 4


def _layer_body(xs_hbm, ro_hbm, col_hbm, norm_hbm, out_hbm,
                rowv, colv, normv, b0, b1, b2, b3,
                g0, g1, g2, g3, s0, s1, s2, s3, out_sp):
    c = lax.axis_index("c")
    s = lax.axis_index("s")
    nfull = TR // CK
    rem = TR - nfull * CK
    bufs = (b0, b1, b2, b3)
    gsems = (g0, g1, g2, g3)
    ssems = (s0, s1, s2, s3)

    def zb(i, _):
        b0[i // 2, pl.ds((i % 2) * 16, 16)] = jnp.zeros((16,), _f32)
        return 0
    lax.fori_loop(0, CK * 2, zb, 0, unroll=8)
    for k in range(nfull):
        pltpu.sync_copy(b0, out_sp.at[pl.ds(s * TR + k * CK, CK), :])
    pltpu.sync_copy(b0.at[pl.ds(0, rem), :],
                    out_sp.at[pl.ds(s * TR + nfull * CK, rem), :])
    plsc.subcore_barrier()

    for t in range(2):
        w = s + t * 16

        def group(g, _):
            base = w * CH + g * GC
            pltpu.sync_copy(ro_hbm.at[pl.ds(c * (EP // CK) + base, GC), :], rowv)
            pltpu.sync_copy(col_hbm.at[pl.ds(base, GC), :], colv)
            pltpu.sync_copy(norm_hbm.at[pl.ds(base, GC), :], normv)

            gcp = [None] * NB
            scp = [None] * NB
            for p in range(NB - 1):
                gcp[p] = pltpu.async_copy(xs_hbm.at[rowv.at[p]], bufs[p], gsems[p])
            for j in range(GC):
                b = j % NB
                gcp[b].wait()
                buf = bufs[b]
                jj = j

                @plsc.parallel_loop(0, CK, unroll=4)
                def edge(k, _buf=buf, _j=jj):
                    sc = plsc.load_gather(normv, [jnp.full((16,), _j, _i32),
                                                  jnp.full((16,), k, _i32)])
                    for q in range(2):
                        _buf[k, pl.ds(q * 16, 16)] = _buf[k, pl.ds(q * 16, 16)] * sc
                scp[b] = pltpu.async_copy(buf, out_sp.at[colv.at[j]],
                                          ssems[b], add=True)
                nj = j + NB - 1
                if nj < GC:
                    nb = nj % NB
                    if scp[nb] is not None:
                        scp[nb].wait()
                        scp[nb] = None
                    gcp[nb] = pltpu.async_copy(
                        xs_hbm.at[rowv.at[nj]], bufs[nb], gsems[nb])
            for x in range(NB):
                if scp[x] is not None:
                    scp[x].wait()
            return 0
        lax.fori_loop(0, CH // GC, group, 0)
    plsc.subcore_barrier()

    for k in range(nfull + 1):
        cnt = CK if k < nfull else rem
        off = s * TR + k * CK
        pltpu.sync_copy(out_sp.at[pl.ds(off, cnt), :], b0.at[pl.ds(0, cnt), :])

        def rl(i, _):
            r = i // 2
            q = (i % 2) * 16
            b0[r, pl.ds(q, 16)] = jnp.maximum(b0[r, pl.ds(q, 16)], 0.0)
            return 0
        lax.fori_loop(0, cnt * 2, rl, 0, unroll=4)
        pltpu.sync_copy(b0.at[pl.ds(0, cnt), :], out_hbm.at[c, pl.ds(off, cnt), :])


_layer_kernel = functools.partial(
    pl.kernel,
    out_type=jax.ShapeDtypeStruct((2, NHS, DW), _f32),
    mesh=_mesh,
    compiler_params=pltpu.CompilerParams(needs_layout_passes=False,
                                         use_tc_tiling_on_sc=False),
    scratch_types=[
        pltpu.VMEM((GC, CK), _i32),
        pltpu.VMEM((GC, CK), _i32),
        pltpu.VMEM((GC, CK), _f32),
        pltpu.VMEM((CK, DW), _f32),
        pltpu.VMEM((CK, DW), _f32),
        pltpu.VMEM((CK, DW), _f32),
        pltpu.VMEM((CK, DW), _f32),
        pltpu.SemaphoreType.DMA,
        pltpu.SemaphoreType.DMA,
        pltpu.SemaphoreType.DMA,
        pltpu.SemaphoreType.DMA,
        pltpu.SemaphoreType.DMA,
        pltpu.SemaphoreType.DMA,
        pltpu.SemaphoreType.DMA,
        pltpu.SemaphoreType.DMA,
        pltpu.VMEM_SHARED((NHS, DW), _f32),
    ],
)(_layer_body)


def _final_body(x0, r1a, r1b, r2a, r2b, w_ref, b_ref, o):
    r1 = jnp.concatenate([r1a[0], r1b[0]], axis=-1)
    r2 = jnp.concatenate([r2a[0], r2b[0]], axis=-1)
    t = (x0[...] + r1 + r2) * (1.0 / 3.0)
    o[...] = lax.dot_general(t, w_ref[...], (((1,), (1,)), ((), ())),
                             preferred_element_type=_f32) + b_ref[...]


def _final(x0, y1, y2, W, b):
    blk = 400
    grid = NN // blk
    half_spec0 = pl.BlockSpec((1, blk, DW), lambda i: (0, i, 0))
    half_spec1 = pl.BlockSpec((1, blk, DW), lambda i: (1, i, 0))
    return pl.pallas_call(
        _final_body,
        grid=(grid,),
        in_specs=[
            pl.BlockSpec((blk, DD), lambda i: (i, 0)),
            half_spec0, half_spec1, half_spec0, half_spec1,
            pl.BlockSpec((DD, DD), lambda i: (0, 0)),
            pl.BlockSpec((1, DD), lambda i: (0, 0)),
        ],
        out_specs=pl.BlockSpec((blk, DD), lambda i: (i, 0)),
        out_shape=jax.ShapeDtypeStruct((NN, DD), _f32),
    )(x0, y1, y1, y2, y2, W, b.reshape(1, DD))


@jax.jit
def kernel(edge_index, edge_weight, emb_users, emb_items, W, b):
    row = edge_index[0].astype(_i32)
    col = edge_index[1].astype(_i32)
    pad = EP - EE
    row3 = jnp.concatenate([row, jnp.zeros((pad,), _i32)]).reshape(NW, CH, CK)
    col3 = jnp.concatenate([col, jnp.full((pad,), NN, _i32)]).reshape(NW, CH, CK)
    ew3 = jnp.concatenate([edge_weight, jnp.zeros((pad,), _f32)]).reshape(NW, CH, CK)

    deg2 = _deg_kernel(col3, ew3).reshape(2, NP_)
    d0 = deg2[0].reshape(NP_ // 128, 128)
    d1 = deg2[1].reshape(NP_ // 128, 128)
    dis = pl.pallas_call(
        _dis_body,
        out_shape=jax.ShapeDtypeStruct((NP_ // 128, 128), _f32),
    )(d0, d1).reshape(NP_)

    row1 = row3.reshape(EP)
    col1 = col3.reshape(EP)
    ew1 = ew3.reshape(EP)
    norm1, lc1f = _norm_kernel(dis, row1, col1, ew1)

    x0 = jnp.concatenate([emb_users, emb_items], axis=0)
    padr = jnp.zeros((NHS - NN, DW), _f32)
    xs0 = jnp.concatenate([x0[:, :DW], padr, x0[:, DW:], padr], axis=0)
    col2 = col1.reshape(EP // CK, CK)
    ro2 = lc1f.reshape(2 * EP // CK, CK)
    norm2 = norm1.reshape(EP // CK, CK)
    y1 = _layer_kernel(xs0, ro2, col2, norm2)
    y2 = _layer_kernel(y1.reshape(2 * NHS, DW), ro2, col2, norm2)

    out = _final(x0, y1, y2, W, b)
    return (out[:NU], emb_users, out[NU:], emb_items)


# speedup vs baseline: 21.3991x; 1.0775x over previous
import functools
import jax
import jax.numpy as jnp
from jax import lax
from jax.experimental import pallas as pl
from jax.experimental.pallas import tpu as pltpu
from jax.experimental.pallas import tpu_sc as plsc

NU = 25000
NI = 25000
NN = NU + NI
EE = 800000
DD = 64
NW = 32
CK = 128
CH = 196
EW_ = CK * CH
EP = NW * EW_
NP_ = 50176
NT = 3136
DW = 32
NHS = 50048
TR = NHS // 16

_mesh = plsc.VectorSubcoreMesh(core_axis_name="c", subcore_axis_name="s")
_f32 = jnp.float32
_i32 = jnp.int32


def _deg_body(col_hbm, ew_hbm, out_hbm, colv, ewv, zbuf,
              d0, d1, d2, d3, deg_sp):
    c = lax.axis_index("c")
    s = lax.axis_index("s")
    w = s * 2 + c
    dsems = (d0, d1, d2, d3)

    def zb(i, _):
        zbuf[pl.ds(i * 16, 16)] = jnp.zeros((16,), _f32)
        return 0
    lax.fori_loop(0, NT // 16, zb, 0, unroll=8)
    pltpu.sync_copy(zbuf, deg_sp.at[pl.ds(s * NT, NT)])
    plsc.subcore_barrier()

    pltpu.sync_copy(col_hbm.at[w], colv)
    pltpu.sync_copy(ew_hbm.at[w], ewv)

    scp = [None] * 4
    for j in range(CH):
        b = j % 4
        if scp[b] is not None:
            scp[b].wait()
        scp[b] = pltpu.async_copy(ewv.at[j], deg_sp.at[colv.at[j]],
                                  dsems[b], add=True)
    for x in range(4):
        if scp[x] is not None:
            scp[x].wait()
    plsc.subcore_barrier()
    pltpu.sync_copy(deg_sp.at[pl.ds(s * NT, NT)], zbuf)
    pltpu.sync_copy(zbuf, out_hbm.at[pl.ds(c * NP_ + s * NT, NT)])


_deg_kernel = functools.partial(
    pl.kernel,
    out_type=jax.ShapeDtypeStruct((2 * NP_,), _f32),
    mesh=_mesh,
    compiler_params=pltpu.CompilerParams(needs_layout_passes=False),
    scratch_types=[
        pltpu.VMEM((CH, CK), _i32),
        pltpu.VMEM((CH, CK), _f32),
        pltpu.VMEM((NT,), _f32),
        pltpu.SemaphoreType.DMA,
        pltpu.SemaphoreType.DMA,
        pltpu.SemaphoreType.DMA,
        pltpu.SemaphoreType.DMA,
        pltpu.VMEM_SHARED((NP_,), _f32),
    ],
)(_deg_body)


def _dis_body(d0, d1, o):
    d = d0[...] + d1[...]
    o[...] = jnp.where(d > 0.0, lax.rsqrt(d), 0.0)


QN = EW_ // 2


def _norm_body(dis_hbm, row_hbm, col_hbm, ew_hbm, norm_hbm, lc_hbm,
               disv, rowq, colq, ewq, normq, lc0q, lc1q):
    c = lax.axis_index("c")
    s = lax.axis_index("s")
    w = s * 2 + c
    pltpu.sync_copy(dis_hbm, disv)
    for h in range(2):
        off = w * EW_ + h * QN
        pltpu.sync_copy(row_hbm.at[pl.ds(off, QN)], rowq)
        pltpu.sync_copy(col_hbm.at[pl.ds(off, QN)], colq)
        pltpu.sync_copy(ew_hbm.at[pl.ds(off, QN)], ewq)

        def step(t, _):
            i = t * 16
            rv = rowq[pl.ds(i, 16)]
            cv = colq[pl.ds(i, 16)]
            ev = ewq[pl.ds(i, 16)]
            a = plsc.load_gather(disv, [rv])
            b = plsc.load_gather(disv, [cv])
            normq[pl.ds(i, 16)] = a * b * ev
            lc0q[pl.ds(i, 16)] = rv
            lc1q[pl.ds(i, 16)] = rv + NHS
            return 0
        lax.fori_loop(0, QN // 16, step, 0, unroll=2)
        pltpu.sync_copy(normq, norm_hbm.at[pl.ds(off, QN)])
        pltpu.sync_copy(lc0q, lc_hbm.at[pl.ds(off, QN)])
        pltpu.sync_copy(lc1q, lc_hbm.at[pl.ds(EP + off, QN)])


_norm_kernel = functools.partial(
    pl.kernel,
    out_type=(jax.ShapeDtypeStruct((EP,), _f32),
              jax.ShapeDtypeStruct((2 * EP,), _i32)),
    mesh=_mesh,
    compiler_params=pltpu.CompilerParams(needs_layout_passes=False),
    scratch_types=[
        pltpu.VMEM((NP_,), _f32),
        pltpu.VMEM((QN,), _i32),
        pltpu.VMEM((QN,), _i32),
        pltpu.VMEM((QN,), _f32),
        pltpu.VMEM((QN,), _f32),
        pltpu.VMEM((QN,), _i32),
        pltpu.VMEM((QN,), _i32),
    ],
)(_norm_body)


GC = 28
NB = 4


def _layer_body(xs_hbm, ro_hbm, col_hbm, norm_hbm, out_hbm,
                rowv, colv, normv, b0, b1, b2, b3,
                g0, g1, g2, g3, s0, s1, s2, s3, out_sp):
    c = lax.axis_index("c")
    s = lax.axis_index("s")
    nfull = TR // CK
    rem = TR - nfull * CK
    bufs = (b0, b1, b2, b3)
    gsems = (g0, g1, g2, g3)
    ssems = (s0, s1, s2, s3)

    def zb(i, _):
        b0[i // 2, pl.ds((i % 2) * 16, 16)] = jnp.zeros((16,), _f32)
        return 0
    lax.fori_loop(0, CK * 2, zb, 0, unroll=8)
    for k in range(nfull):
        pltpu.sync_copy(b0, out_sp.at[pl.ds(s * TR + k * CK, CK), :])
    pltpu.sync_copy(b0.at[pl.ds(0, rem), :],
                    out_sp.at[pl.ds(s * TR + nfull * CK, rem), :])
    plsc.subcore_barrier()

    for t in range(2):
        w = s + t * 16

        def group(g, _):
            base = w * CH + g * GC
            pltpu.sync_copy(ro_hbm.at[pl.ds(c * (EP // CK) + base, GC), :], rowv)
            pltpu.sync_copy(col_hbm.at[pl.ds(base, GC), :], colv)
            pltpu.sync_copy(norm_hbm.at[pl.ds(base, GC), :], normv)

            gcp = [None] * NB
            scp = [None] * NB
            for p in range(NB - 1):
                gcp[p] = pltpu.async_copy(xs_hbm.at[rowv.at[p]], bufs[p], gsems[p])
            for j in range(GC):
                b = j % NB
                gcp[b].wait()
                buf = bufs[b]
                jj = j

                @plsc.parallel_loop(0, CK, unroll=4)
                def edge(k, _buf=buf, _j=jj):
                    sc = plsc.load_gather(normv, [jnp.full((16,), _j, _i32),
                                                  jnp.full((16,), k, _i32)])
                    for q in range(2):
                        _buf[k, pl.ds(q * 16, 16)] = _buf[k, pl.ds(q * 16, 16)] * sc
                scp[b] = pltpu.async_copy(buf, out_sp.at[colv.at[j]],
                                          ssems[b], add=True)
                nj = j + NB - 1
                if nj < GC:
                    nb = nj % NB
                    if scp[nb] is not None:
                        scp[nb].wait()
                        scp[nb] = None
                    gcp[nb] = pltpu.async_copy(
                        xs_hbm.at[rowv.at[nj]], bufs[nb], gsems[nb])
            for x in range(NB):
                if scp[x] is not None:
                    scp[x].wait()
            return 0
        lax.fori_loop(0, CH // GC, group, 0)
    plsc.subcore_barrier()

    for k in range(nfull + 1):
        cnt = CK if k < nfull else rem
        off = s * TR + k * CK
        pltpu.sync_copy(out_sp.at[pl.ds(off, cnt), :], b0.at[pl.ds(0, cnt), :])

        def rl(i, _):
            r = i // 2
            q = (i % 2) * 16
            b0[r, pl.ds(q, 16)] = jnp.maximum(b0[r, pl.ds(q, 16)], 0.0)
            return 0
        lax.fori_loop(0, cnt * 2, rl, 0, unroll=4)
        pltpu.sync_copy(b0.at[pl.ds(0, cnt), :], out_hbm.at[c, pl.ds(off, cnt), :])


_layer_kernel = functools.partial(
    pl.kernel,
    out_type=jax.ShapeDtypeStruct((2, NHS, DW), _f32),
    mesh=_mesh,
    compiler_params=pltpu.CompilerParams(needs_layout_passes=False,
                                         use_tc_tiling_on_sc=False),
    scratch_types=[
        pltpu.VMEM((GC, CK), _i32),
        pltpu.VMEM((GC, CK), _i32),
        pltpu.VMEM((GC, CK), _f32),
        pltpu.VMEM((CK, DW), _f32),
        pltpu.VMEM((CK, DW), _f32),
        pltpu.VMEM((CK, DW), _f32),
        pltpu.VMEM((CK, DW), _f32),
        pltpu.SemaphoreType.DMA,
        pltpu.SemaphoreType.DMA,
        pltpu.SemaphoreType.DMA,
        pltpu.SemaphoreType.DMA,
        pltpu.SemaphoreType.DMA,
        pltpu.SemaphoreType.DMA,
        pltpu.SemaphoreType.DMA,
        pltpu.SemaphoreType.DMA,
        pltpu.VMEM_SHARED((NHS, DW), _f32),
    ],
)(_layer_body)


def _final_body(x0, r1a, r1b, r2a, r2b, w_ref, b_ref, o):
    r1 = jnp.concatenate([r1a[0], r1b[0]], axis=-1)
    r2 = jnp.concatenate([r2a[0], r2b[0]], axis=-1)
    t = (x0[...] + r1 + r2) * (1.0 / 3.0)
    o[...] = lax.dot_general(t, w_ref[...], (((1,), (1,)), ((), ())),
                             preferred_element_type=_f32) + b_ref[...]


def _final(x0, y1, y2, W, b):
    blk = 400
    grid = NN // blk
    half_spec0 = pl.BlockSpec((1, blk, DW), lambda i: (0, i, 0))
    half_spec1 = pl.BlockSpec((1, blk, DW), lambda i: (1, i, 0))
    return pl.pallas_call(
        _final_body,
        grid=(grid,),
        in_specs=[
            pl.BlockSpec((blk, DD), lambda i: (i, 0)),
            half_spec0, half_spec1, half_spec0, half_spec1,
            pl.BlockSpec((DD, DD), lambda i: (0, 0)),
            pl.BlockSpec((1, DD), lambda i: (0, 0)),
        ],
        out_specs=pl.BlockSpec((blk, DD), lambda i: (i, 0)),
        out_shape=jax.ShapeDtypeStruct((NN, DD), _f32),
    )(x0, y1, y1, y2, y2, W, b.reshape(1, DD))


@jax.jit
def kernel(edge_index, edge_weight, emb_users, emb_items, W, b):
    row = edge_index[0].astype(_i32)
    col = edge_index[1].astype(_i32)
    pad = EP - EE
    row3 = jnp.concatenate([row, jnp.zeros((pad,), _i32)]).reshape(NW, CH, CK)
    col3 = jnp.concatenate([col, jnp.full((pad,), NN, _i32)]).reshape(NW, CH, CK)
    ew3 = jnp.concatenate([edge_weight, jnp.zeros((pad,), _f32)]).reshape(NW, CH, CK)

    deg2 = _deg_kernel(col3, ew3).reshape(2, NP_)
    d0 = deg2[0].reshape(NP_ // 128, 128)
    d1 = deg2[1].reshape(NP_ // 128, 128)
    dis = pl.pallas_call(
        _dis_body,
        out_shape=jax.ShapeDtypeStruct((NP_ // 128, 128), _f32),
    )(d0, d1).reshape(NP_)

    row1 = row3.reshape(EP)
    col1 = col3.reshape(EP)
    ew1 = ew3.reshape(EP)
    norm1, lc1f = _norm_kernel(dis, row1, col1, ew1)

    x0 = jnp.concatenate([emb_users, emb_items], axis=0)
    padr = jnp.zeros((NHS - NN, DW), _f32)
    xs0 = jnp.concatenate([x0[:, :DW], padr, x0[:, DW:], padr], axis=0)
    col2 = col1.reshape(EP // CK, CK)
    ro2 = lc1f.reshape(2 * EP // CK, CK)
    norm2 = norm1.reshape(EP // CK, CK)
    y1 = _layer_kernel(xs0, ro2, col2, norm2)
    y2 = _layer_kernel(y1.reshape(2 * NHS, DW), ro2, col2, norm2)

    out = _final(x0, y1, y2, W, b)
    return (out[:NU], emb_users, out[NU:], emb_items)
